# baseline (speedup 1.0000x reference)
import jax
import jax.numpy as jnp
from jax import lax
from jax.experimental import pallas as pl
from jax.experimental.pallas import tpu as pltpu
from jax.experimental.pallas import tpu_sc as plsc

N = 100000
D = 128
G = 256
NC = 2
NS = 16
NW = NC * NS

CHUNK = 3136
LAST = N - (NW - 1) * CHUNK
BLK = 224
NBLK_FULL = CHUNK // BLK
NBLK_LAST = LAST // BLK
TAIL_LAST = LAST - NBLK_LAST * BLK

NEG_INF = float("-inf")


def _seg_partials(feat_hbm, ids_hbm, part_hbm, idsv, buf0, buf1, acc, sem0, sem1):
    c = lax.axis_index("c")
    s = lax.axis_index("s")
    w = s * NC + c
    base = w * CHUNK
    is_last = w == NW - 1

    neg = jnp.full((16,), NEG_INF, jnp.float32)

    def init_body(g, carry):
        for f in range(D // 16):
            acc[g, pl.ds(f * 16, 16)] = neg
        return carry

    lax.fori_loop(0, G, init_body, jnp.int32(0))

    def process_block(bufref, ids_off, rows, carry):
        def grp(tt, carry):
            cur, ms = carry
            r0 = tt * 16
            idvec = idsv[pl.ds(ids_off + r0, 16)]
            for j in range(16):
                g = idvec[j]
                same = g == cur
                new_ms = []
                for f in range(D // 16):
                    x = bufref[r0 + j, pl.ds(f * 16, 16)]
                    m = jnp.maximum(jnp.where(same, ms[f], neg), x)
                    acc[g, pl.ds(f * 16, 16)] = m
                    new_ms.append(m)
                ms = tuple(new_ms)
                cur = g
            return (cur, ms)

        return lax.fori_loop(0, rows // 16, grp, carry)

    def run(nblk, tail):
        nrows = nblk * BLK + tail
        pltpu.sync_copy(ids_hbm.at[pl.ds(base, nrows)], idsv.at[pl.ds(0, nrows)])

        pltpu.async_copy(feat_hbm.at[pl.ds(base, BLK), :], buf0, sem0)

        npair = nblk // 2

        def pair_body(t, carry):
            b0 = 2 * t
            pltpu.make_async_copy(feat_hbm.at[pl.ds(base, BLK), :], buf0, sem0).wait()
            pltpu.async_copy(
                feat_hbm.at[pl.ds(base + (b0 + 1) * BLK, BLK), :], buf1, sem1
            )
            carry = process_block(buf0, b0 * BLK, BLK, carry)
            pltpu.make_async_copy(feat_hbm.at[pl.ds(base, BLK), :], buf1, sem1).wait()

            @pl.when(b0 + 2 < nblk)
            def _():
                pltpu.async_copy(
                    feat_hbm.at[pl.ds(base + (b0 + 2) * BLK, BLK), :], buf0, sem0
                )

            carry = process_block(buf1, (b0 + 1) * BLK, BLK, carry)
            return carry

        carry0 = (jnp.int32(-1), (neg,) * (D // 16))
        carry = lax.fori_loop(0, npair, pair_body, carry0)

        if tail:
            pltpu.sync_copy(
                feat_hbm.at[pl.ds(base + nblk * BLK, tail), :],
                buf0.at[pl.ds(0, tail), :],
            )
            process_block(buf0, nblk * BLK, tail, carry)

    @pl.when(jnp.logical_not(is_last))
    def _():
        run(NBLK_FULL, 0)

    @pl.when(is_last)
    def _():
        run(NBLK_LAST, TAIL_LAST)

    pltpu.sync_copy(acc, part_hbm.at[w])


_mesh = plsc.VectorSubcoreMesh(
    core_axis_name="c", subcore_axis_name="s", num_cores=NC, num_subcores=NS
)

_phase1 = pl.kernel(
    _seg_partials,
    out_type=jax.ShapeDtypeStruct((NW, G, D), jnp.float32),
    mesh=_mesh,
    scratch_types=[
        pltpu.VMEM((CHUNK,), jnp.int32),
        pltpu.VMEM((BLK, D), jnp.float32),
        pltpu.VMEM((BLK, D), jnp.float32),
        pltpu.VMEM((G, D), jnp.float32),
        pltpu.SemaphoreType.DMA,
        pltpu.SemaphoreType.DMA,
    ],
)


def _combine_body(parts_ref, out_ref):
    out_ref[...] = jnp.max(parts_ref[...], axis=0)


_combine = pl.pallas_call(
    _combine_body,
    out_shape=jax.ShapeDtypeStruct((G, D), jnp.float32),
)


@jax.jit
def _impl(feat, ids):
    partials = _phase1(feat, ids)
    return _combine(partials)


def kernel(feat0, segment_ids):
    return _impl(feat0[..., 0], segment_ids)


# speedup vs baseline: 2.2697x; 2.2697x over previous
import jax
import jax.numpy as jnp
from jax import lax
from jax.experimental import pallas as pl
from jax.experimental.pallas import tpu as pltpu
from jax.experimental.pallas import tpu_sc as plsc

N = 100000
D = 128
G = 256
NC = 2
NS = 16
NW = NC * NS

CHUNK = 3136
LAST = N - (NW - 1) * CHUNK
BLK = 224
NBLK_FULL = CHUNK // BLK
NBLK_LAST = LAST // BLK
TAIL_LAST = LAST - NBLK_LAST * BLK

NEG_INF = float("-inf")


def _seg_partials(feat_hbm, ids_hbm, part_hbm, idsv, buf0, buf1, acc, sem0, sem1):
    c = lax.axis_index("c")
    s = lax.axis_index("s")
    w = s * NC + c
    base = w * CHUNK
    is_last = w == NW - 1

    neg = jnp.full((16,), NEG_INF, jnp.float32)

    def init_body(g, carry):
        for f in range(D // 16):
            acc[g, pl.ds(f * 16, 16)] = neg
        return carry

    lax.fori_loop(0, G, init_body, jnp.int32(0))

    cols = [lax.iota(jnp.int32, 16) + f * 16 for f in range(D // 16)]

    def process_block(bufref, ids_off, rows, carry):
        def grp(tt, carry):
            cur_v, ms = carry
            r0 = tt * 16
            for j in range(16):
                ridx = jnp.full((16,), ids_off + r0 + j, jnp.int32)
                g_v = plsc.load_gather(idsv, [ridx])
                xs = [bufref[r0 + j, pl.ds(f * 16, 16)] for f in range(D // 16)]
                same = g_v == cur_v
                new_ms = []
                for f in range(D // 16):
                    m = jnp.maximum(jnp.where(same, ms[f], neg), xs[f])
                    new_ms.append(m)
                for f in range(D // 16):
                    plsc.store_scatter(acc, [g_v, cols[f]], new_ms[f])
                ms = tuple(new_ms)
                cur_v = g_v
            return (cur_v, ms)

        return lax.fori_loop(0, rows // 16, grp, carry)

    def run(nblk, tail):
        nrows = nblk * BLK + tail
        pltpu.sync_copy(ids_hbm.at[pl.ds(base, nrows)], idsv.at[pl.ds(0, nrows)])

        pltpu.async_copy(feat_hbm.at[pl.ds(base, BLK), :], buf0, sem0)

        npair = nblk // 2

        def pair_body(t, carry):
            b0 = 2 * t
            pltpu.make_async_copy(feat_hbm.at[pl.ds(base, BLK), :], buf0, sem0).wait()
            pltpu.async_copy(
                feat_hbm.at[pl.ds(base + (b0 + 1) * BLK, BLK), :], buf1, sem1
            )
            carry = process_block(buf0, b0 * BLK, BLK, carry)
            pltpu.make_async_copy(feat_hbm.at[pl.ds(base, BLK), :], buf1, sem1).wait()

            @pl.when(b0 + 2 < nblk)
            def _():
                pltpu.async_copy(
                    feat_hbm.at[pl.ds(base + (b0 + 2) * BLK, BLK), :], buf0, sem0
                )

            carry = process_block(buf1, (b0 + 1) * BLK, BLK, carry)
            return carry

        carry0 = (jnp.full((16,), -1, jnp.int32), (neg,) * (D // 16))
        carry = lax.fori_loop(0, npair, pair_body, carry0)

        if tail:
            pltpu.sync_copy(
                feat_hbm.at[pl.ds(base + nblk * BLK, tail), :],
                buf0.at[pl.ds(0, tail), :],
            )
            process_block(buf0, nblk * BLK, tail, carry)

    @pl.when(jnp.logical_not(is_last))
    def _():
        run(NBLK_FULL, 0)

    @pl.when(is_last)
    def _():
        run(NBLK_LAST, TAIL_LAST)

    pltpu.sync_copy(acc, part_hbm.at[w])


_mesh = plsc.VectorSubcoreMesh(
    core_axis_name="c", subcore_axis_name="s", num_cores=NC, num_subcores=NS
)

_phase1 = pl.kernel(
    _seg_partials,
    out_type=jax.ShapeDtypeStruct((NW, G, D), jnp.float32),
    mesh=_mesh,
    compiler_params=pltpu.CompilerParams(needs_layout_passes=False),
    scratch_types=[
        pltpu.VMEM((CHUNK,), jnp.int32),
        pltpu.VMEM((BLK, D), jnp.float32),
        pltpu.VMEM((BLK, D), jnp.float32),
        pltpu.VMEM((G, D), jnp.float32),
        pltpu.SemaphoreType.DMA,
        pltpu.SemaphoreType.DMA,
    ],
)


def _combine_body(parts_ref, out_ref):
    out_ref[...] = jnp.max(parts_ref[...], axis=0)


_combine = pl.pallas_call(
    _combine_body,
    out_shape=jax.ShapeDtypeStruct((G, D), jnp.float32),
)


@jax.jit
def _impl(feat, ids):
    partials = _phase1(feat, ids)
    return _combine(partials)


def kernel(feat0, segment_ids):
    return _impl(feat0[..., 0], segment_ids)


# speedup vs baseline: 2.5164x; 1.1087x over previous
import jax
import jax.numpy as jnp
from jax import lax
from jax.experimental import pallas as pl
from jax.experimental.pallas import tpu as pltpu
from jax.experimental.pallas import tpu_sc as plsc

N = 100000
D = 128
G = 256
NC = 2
NS = 16
NW = NC * NS

CHUNK = 3136
LAST = N - (NW - 1) * CHUNK
BLK = 224
NBLK_FULL = CHUNK // BLK
NBLK_LAST = LAST // BLK
TAIL_LAST = LAST - NBLK_LAST * BLK

NEG_INF = float("-inf")


def _seg_partials(feat_hbm, ids_hbm, part_hbm, idsv, buf0, buf1, acc, sem0, sem1):
    c = lax.axis_index("c")
    s = lax.axis_index("s")
    w = s * NC + c
    base = w * CHUNK
    is_last = w == NW - 1

    neg = jnp.full((16,), NEG_INF, jnp.float32)

    def init_body(g, carry):
        for f in range(D // 16):
            acc[g, pl.ds(f * 16, 16)] = neg
        return carry

    lax.fori_loop(0, G, init_body, jnp.int32(0))

    cols = [lax.iota(jnp.int32, 16) + f * 16 for f in range(D // 16)]

    def tree_max(vs):
        vs = list(vs)
        while len(vs) > 1:
            nxt = [jnp.maximum(vs[i], vs[i + 1]) for i in range(0, len(vs) - 1, 2)]
            if len(vs) % 2:
                nxt.append(vs[-1])
            vs = nxt
        return vs[0]

    def flush(carry):
        cur_v, ms = carry
        for f in range(D // 16):
            plsc.store_scatter(acc, [cur_v, cols[f]], ms[f])

    def process_block(bufref, ids_off, rows, carry):
        def grp(tt, carry):
            cur_v, ms = carry
            r0 = tt * 16
            idvec = idsv[pl.ds(ids_off + r0, 16)]
            uniform = jnp.all(idvec == cur_v)

            def fast(carry):
                cur_v, ms = carry
                new_ms = []
                for f in range(D // 16):
                    xs = [bufref[r0 + j, pl.ds(f * 16, 16)] for j in range(16)]
                    new_ms.append(jnp.maximum(ms[f], tree_max(xs)))
                return (cur_v, tuple(new_ms))

            def slow(carry):
                cur_v, ms = carry
                flush(carry)
                for j in range(16):
                    ridx = jnp.full((16,), ids_off + r0 + j, jnp.int32)
                    g_v = plsc.load_gather(idsv, [ridx])
                    xs = [bufref[r0 + j, pl.ds(f * 16, 16)] for f in range(D // 16)]
                    same = g_v == cur_v
                    new_ms = [
                        jnp.maximum(jnp.where(same, ms[f], neg), xs[f])
                        for f in range(D // 16)
                    ]
                    for f in range(D // 16):
                        plsc.store_scatter(acc, [g_v, cols[f]], new_ms[f])
                    ms = tuple(new_ms)
                    cur_v = g_v
                return (cur_v, ms)

            return lax.cond(uniform, fast, slow, carry)

        return lax.fori_loop(0, rows // 16, grp, carry)

    def run(nblk, tail):
        nrows = nblk * BLK + tail
        pltpu.sync_copy(ids_hbm.at[pl.ds(base, nrows)], idsv.at[pl.ds(0, nrows)])

        pltpu.async_copy(feat_hbm.at[pl.ds(base, BLK), :], buf0, sem0)

        npair = nblk // 2

        def pair_body(t, carry):
            b0 = 2 * t
            pltpu.make_async_copy(feat_hbm.at[pl.ds(base, BLK), :], buf0, sem0).wait()
            pltpu.async_copy(
                feat_hbm.at[pl.ds(base + (b0 + 1) * BLK, BLK), :], buf1, sem1
            )
            carry = process_block(buf0, b0 * BLK, BLK, carry)
            pltpu.make_async_copy(feat_hbm.at[pl.ds(base, BLK), :], buf1, sem1).wait()

            @pl.when(b0 + 2 < nblk)
            def _():
                pltpu.async_copy(
                    feat_hbm.at[pl.ds(base + (b0 + 2) * BLK, BLK), :], buf0, sem0
                )

            carry = process_block(buf1, (b0 + 1) * BLK, BLK, carry)
            return carry

        g0 = plsc.load_gather(idsv, [jnp.zeros((16,), jnp.int32)])
        carry0 = (g0, (neg,) * (D // 16))
        carry = lax.fori_loop(0, npair, pair_body, carry0)

        if tail:
            pltpu.sync_copy(
                feat_hbm.at[pl.ds(base + nblk * BLK, tail), :],
                buf0.at[pl.ds(0, tail), :],
            )
            carry = process_block(buf0, nblk * BLK, tail, carry)

        flush(carry)

    @pl.when(jnp.logical_not(is_last))
    def _():
        run(NBLK_FULL, 0)

    @pl.when(is_last)
    def _():
        run(NBLK_LAST, TAIL_LAST)

    pltpu.sync_copy(acc, part_hbm.at[w])


_mesh = plsc.VectorSubcoreMesh(
    core_axis_name="c", subcore_axis_name="s", num_cores=NC, num_subcores=NS
)

_phase1 = pl.kernel(
    _seg_partials,
    out_type=jax.ShapeDtypeStruct((NW, G, D), jnp.float32),
    mesh=_mesh,
    compiler_params=pltpu.CompilerParams(needs_layout_passes=False),
    scratch_types=[
        pltpu.VMEM((CHUNK,), jnp.int32),
        pltpu.VMEM((BLK, D), jnp.float32),
        pltpu.VMEM((BLK, D), jnp.float32),
        pltpu.VMEM((G, D), jnp.float32),
        pltpu.SemaphoreType.DMA,
        pltpu.SemaphoreType.DMA,
    ],
)


def _combine_body(parts_ref, out_ref):
    out_ref[...] = jnp.max(parts_ref[...], axis=0)


_combine = pl.pallas_call(
    _combine_body,
    out_shape=jax.ShapeDtypeStruct((G, D), jnp.float32),
)


@jax.jit
def _impl(feat, ids):
    partials = _phase1(feat, ids)
    return _combine(partials)


def kernel(feat0, segment_ids):
    return _impl(feat0[..., 0], segment_ids)


# speedup vs baseline: 2.5180x; 1.0006x over previous
import jax
import jax.numpy as jnp
from jax import lax
from jax.experimental import pallas as pl
from jax.experimental.pallas import tpu as pltpu
from jax.experimental.pallas import tpu_sc as plsc

N = 100000
D = 128
G = 256
NC = 2
NS = 16
NW = NC * NS

CHUNK = 3136
LAST = N - (NW - 1) * CHUNK
BLK = 224
NBLK_FULL = CHUNK // BLK
NBLK_LAST = LAST // BLK
TAIL_LAST = LAST - NBLK_LAST * BLK

NEG_INF = float("-inf")


def _seg_partials(feat_hbm, ids_hbm, part_hbm, idsv, buf0, buf1, acc, sem0, sem1):
    c = lax.axis_index("c")
    s = lax.axis_index("s")
    w = s * NC + c
    base = w * CHUNK
    is_last = w == NW - 1

    neg = jnp.full((16,), NEG_INF, jnp.float32)

    def init_body(g, carry):
        for f in range(D // 16):
            acc[g, pl.ds(f * 16, 16)] = neg
        return carry

    lax.fori_loop(0, G, init_body, jnp.int32(0))

    cols = [lax.iota(jnp.int32, 16) + f * 16 for f in range(D // 16)]

    def tree_max(vs):
        vs = list(vs)
        while len(vs) > 1:
            nxt = [jnp.maximum(vs[i], vs[i + 1]) for i in range(0, len(vs) - 1, 2)]
            if len(vs) % 2:
                nxt.append(vs[-1])
            vs = nxt
        return vs[0]

    def flush(carry):
        cur_v, ms = carry
        for f in range(D // 16):
            plsc.store_scatter(acc, [cur_v, cols[f]], ms[f])

    def process_block(bufref, ids_off, rows, carry):
        def grp(tt, carry):
            cur_v, ms = carry
            r0 = tt * 16
            idvec = idsv[pl.ds(ids_off + r0, 16)]
            uniform = jnp.max(jnp.bitwise_xor(idvec, cur_v)) == 0

            def fast(carry):
                cur_v, ms = carry
                new_ms = []
                for f in range(D // 16):
                    xs = [bufref[r0 + j, pl.ds(f * 16, 16)] for j in range(16)]
                    new_ms.append(jnp.maximum(ms[f], tree_max(xs)))
                return (cur_v, tuple(new_ms))

            def slow(carry):
                cur_v, ms = carry
                flush(carry)
                for j in range(16):
                    ridx = jnp.full((16,), ids_off + r0 + j, jnp.int32)
                    g_v = plsc.load_gather(idsv, [ridx])
                    xs = [bufref[r0 + j, pl.ds(f * 16, 16)] for f in range(D // 16)]
                    same = g_v == cur_v
                    new_ms = [
                        jnp.maximum(jnp.where(same, ms[f], neg), xs[f])
                        for f in range(D // 16)
                    ]
                    for f in range(D // 16):
                        plsc.store_scatter(acc, [g_v, cols[f]], new_ms[f])
                    ms = tuple(new_ms)
                    cur_v = g_v
                return (cur_v, ms)

            return lax.cond(uniform, fast, slow, carry)

        return lax.fori_loop(0, rows // 16, grp, carry)

    def run(nblk, tail):
        nrows = nblk * BLK + tail
        pltpu.sync_copy(ids_hbm.at[pl.ds(base, nrows)], idsv.at[pl.ds(0, nrows)])

        pltpu.async_copy(feat_hbm.at[pl.ds(base, BLK), :], buf0, sem0)

        npair = nblk // 2

        def pair_body(t, carry):
            b0 = 2 * t
            pltpu.make_async_copy(feat_hbm.at[pl.ds(base, BLK), :], buf0, sem0).wait()
            pltpu.async_copy(
                feat_hbm.at[pl.ds(base + (b0 + 1) * BLK, BLK), :], buf1, sem1
            )
            carry = process_block(buf0, b0 * BLK, BLK, carry)
            pltpu.make_async_copy(feat_hbm.at[pl.ds(base, BLK), :], buf1, sem1).wait()

            @pl.when(b0 + 2 < nblk)
            def _():
                pltpu.async_copy(
                    feat_hbm.at[pl.ds(base + (b0 + 2) * BLK, BLK), :], buf0, sem0
                )

            carry = process_block(buf1, (b0 + 1) * BLK, BLK, carry)
            return carry

        g0 = plsc.load_gather(idsv, [jnp.zeros((16,), jnp.int32)])
        carry0 = (g0, (neg,) * (D // 16))
        carry = lax.fori_loop(0, npair, pair_body, carry0)

        if tail:
            pltpu.sync_copy(
                feat_hbm.at[pl.ds(base + nblk * BLK, tail), :],
                buf0.at[pl.ds(0, tail), :],
            )
            carry = process_block(buf0, nblk * BLK, tail, carry)

        flush(carry)

    @pl.when(jnp.logical_not(is_last))
    def _():
        run(NBLK_FULL, 0)

    @pl.when(is_last)
    def _():
        run(NBLK_LAST, TAIL_LAST)

    pltpu.sync_copy(acc, part_hbm.at[w])


_mesh = plsc.VectorSubcoreMesh(
    core_axis_name="c", subcore_axis_name="s", num_cores=NC, num_subcores=NS
)

_phase1 = pl.kernel(
    _seg_partials,
    out_type=jax.ShapeDtypeStruct((NW, G, D), jnp.float32),
    mesh=_mesh,
    compiler_params=pltpu.CompilerParams(needs_layout_passes=False),
    scratch_types=[
        pltpu.VMEM((CHUNK,), jnp.int32),
        pltpu.VMEM((BLK, D), jnp.float32),
        pltpu.VMEM((BLK, D), jnp.float32),
        pltpu.VMEM((G, D), jnp.float32),
        pltpu.SemaphoreType.DMA,
        pltpu.SemaphoreType.DMA,
    ],
)


def _combine_body(parts_ref, out_ref):
    out_ref[...] = jnp.max(parts_ref[...], axis=0)


_combine = pl.pallas_call(
    _combine_body,
    out_shape=jax.ShapeDtypeStruct((G, D), jnp.float32),
)


@jax.jit
def _impl(feat, ids):
    partials = _phase1(feat, ids)
    return _combine(partials)


def kernel(feat0, segment_ids):
    return _impl(feat0[..., 0], segment_ids)


# speedup vs baseline: 2.5472x; 1.0116x over previous
import jax
import jax.numpy as jnp
from jax import lax
from jax.experimental import pallas as pl
from jax.experimental.pallas import tpu as pltpu
from jax.experimental.pallas import tpu_sc as plsc

N = 100000
D = 128
G = 256
NC = 2
NS = 16
NW = NC * NS

CHUNK = 3136
LAST = N - (NW - 1) * CHUNK
BLK = 224
NBLK_FULL = CHUNK // BLK
NBLK_LAST = LAST // BLK
TAIL_LAST = LAST - NBLK_LAST * BLK

NEG_INF = float("-inf")


def _seg_partials(feat_hbm, ids_hbm, part_hbm, idsv, buf0, buf1, acc, sem0, sem1):
    c = lax.axis_index("c")
    s = lax.axis_index("s")
    w = s * NC + c
    base = w * CHUNK
    is_last = w == NW - 1

    neg = jnp.full((16,), NEG_INF, jnp.float32)

    def init_body(g, carry):
        for f in range(D // 16):
            acc[g, pl.ds(f * 16, 16)] = neg
        return carry

    lax.fori_loop(0, G, init_body, jnp.int32(0))

    cols = [lax.iota(jnp.int32, 16) + f * 16 for f in range(D // 16)]

    def tree_max(vs):
        vs = list(vs)
        while len(vs) > 1:
            nxt = [jnp.maximum(vs[i], vs[i + 1]) for i in range(0, len(vs) - 1, 2)]
            if len(vs) % 2:
                nxt.append(vs[-1])
            vs = nxt
        return vs[0]

    def flush(carry):
        cur_v, ms = carry
        for f in range(D // 16):
            plsc.store_scatter(acc, [cur_v, cols[f]], ms[f])

    def process_block(bufref, ids_off, rows, carry):
        def grp(tt, carry):
            cur_v, ms, uni = carry
            r0 = tt * 16
            idvec_n = idsv[pl.ds(ids_off + r0 + 16, 16)]

            def fast(carry):
                cur_v, ms, _ = carry
                uni_n = jnp.max(jnp.bitwise_xor(idvec_n, cur_v)) == 0
                new_ms = []
                for f in range(D // 16):
                    xs = [bufref[r0 + j, pl.ds(f * 16, 16)] for j in range(16)]
                    new_ms.append(jnp.maximum(ms[f], tree_max(xs)))
                return (cur_v, tuple(new_ms), uni_n)

            def slow(carry):
                cur_v, ms, _ = carry
                flush((cur_v, ms))
                for j in range(16):
                    ridx = jnp.full((16,), ids_off + r0 + j, jnp.int32)
                    g_v = plsc.load_gather(idsv, [ridx])
                    xs = [bufref[r0 + j, pl.ds(f * 16, 16)] for f in range(D // 16)]
                    same = g_v == cur_v
                    new_ms = [
                        jnp.maximum(jnp.where(same, ms[f], neg), xs[f])
                        for f in range(D // 16)
                    ]
                    for f in range(D // 16):
                        plsc.store_scatter(acc, [g_v, cols[f]], new_ms[f])
                    ms = tuple(new_ms)
                    cur_v = g_v
                uni_n = jnp.max(jnp.bitwise_xor(idvec_n, cur_v)) == 0
                return (cur_v, tuple(ms), uni_n)

            return lax.cond(uni, fast, slow, carry)

        return lax.fori_loop(0, rows // 16, grp, carry)

    def run(nblk, tail):
        nrows = nblk * BLK + tail
        pltpu.sync_copy(ids_hbm.at[pl.ds(base, nrows)], idsv.at[pl.ds(0, nrows)])

        pltpu.async_copy(feat_hbm.at[pl.ds(base, BLK), :], buf0, sem0)

        npair = nblk // 2

        def pair_body(t, carry):
            b0 = 2 * t
            pltpu.make_async_copy(feat_hbm.at[pl.ds(base, BLK), :], buf0, sem0).wait()
            pltpu.async_copy(
                feat_hbm.at[pl.ds(base + (b0 + 1) * BLK, BLK), :], buf1, sem1
            )
            carry = process_block(buf0, b0 * BLK, BLK, carry)
            pltpu.make_async_copy(feat_hbm.at[pl.ds(base, BLK), :], buf1, sem1).wait()

            @pl.when(b0 + 2 < nblk)
            def _():
                pltpu.async_copy(
                    feat_hbm.at[pl.ds(base + (b0 + 2) * BLK, BLK), :], buf0, sem0
                )

            carry = process_block(buf1, (b0 + 1) * BLK, BLK, carry)
            return carry

        g0 = plsc.load_gather(idsv, [jnp.zeros((16,), jnp.int32)])
        idvec0 = idsv[pl.ds(0, 16)]
        uni0 = jnp.max(jnp.bitwise_xor(idvec0, g0)) == 0
        carry0 = (g0, (neg,) * (D // 16), uni0)
        carry = lax.fori_loop(0, npair, pair_body, carry0)

        if tail:
            pltpu.sync_copy(
                feat_hbm.at[pl.ds(base + nblk * BLK, tail), :],
                buf0.at[pl.ds(0, tail), :],
            )
            carry = process_block(buf0, nblk * BLK, tail, carry)

        flush((carry[0], carry[1]))

    @pl.when(jnp.logical_not(is_last))
    def _():
        run(NBLK_FULL, 0)

    @pl.when(is_last)
    def _():
        run(NBLK_LAST, TAIL_LAST)

    pltpu.sync_copy(acc, part_hbm.at[w])


_mesh = plsc.VectorSubcoreMesh(
    core_axis_name="c", subcore_axis_name="s", num_cores=NC, num_subcores=NS
)

_phase1 = pl.kernel(
    _seg_partials,
    out_type=jax.ShapeDtypeStruct((NW, G, D), jnp.float32),
    mesh=_mesh,
    compiler_params=pltpu.CompilerParams(needs_layout_passes=False),
    scratch_types=[
        pltpu.VMEM((CHUNK + 16,), jnp.int32),
        pltpu.VMEM((BLK, D), jnp.float32),
        pltpu.VMEM((BLK, D), jnp.float32),
        pltpu.VMEM((G, D), jnp.float32),
        pltpu.SemaphoreType.DMA,
        pltpu.SemaphoreType.DMA,
    ],
)


def _combine_body(parts_ref, out_ref):
    out_ref[...] = jnp.max(parts_ref[...], axis=0)


_combine = pl.pallas_call(
    _combine_body,
    out_shape=jax.ShapeDtypeStruct((G, D), jnp.float32),
)


@jax.jit
def _impl(feat, ids):
    partials = _phase1(feat, ids)
    return _combine(partials)


def kernel(feat0, segment_ids):
    return _impl(feat0[..., 0], segment_ids)
